# Initial kernel scaffold; baseline (speedup 1.0000x reference)
import jax
import jax.numpy as jnp
from jax.experimental import pallas as pl


def kernel(user_indices, cont, genres, cast, comp, Eu, Eg, Ecast, Ecomp, Wc0, bc0, gc0, btc0, Wc1, bc1, gc1, btc1, Wm0, bm0, gm0, btm0, Wm1, bm1, gm1, btm1, Wb0, bb0, gb0, btb0, Wb1, bb1, gb1, btb1, Wa, ba):
    raise NotImplementedError("write your pallas kernel here")


# speedup vs baseline: 2.6041x; 2.6041x over previous
import functools

import jax
import jax.numpy as jnp
from jax import lax
from jax.experimental import pallas as pl
from jax.experimental.pallas import tpu as pltpu
from jax.experimental.pallas import tpu_sc as plsc

B = 16384
NC, NS = 2, 16
NW = NC * NS
BW = B // NW

G_BAG = 20
K_BAG = 50
G_SUB = 32
K_SUB = 16
G_CH = 80
K_CH = 100
G_ITER = BW // G_SUB
K_ITER = BW // K_SUB


def _sc_body(uidx, cidx, gidx, kidx, Eu, Ecomp, Eg, Ecast,
             feat,
             uidx_v, cidx_v, gidx_v, kidx_v,
             urows_v, crows_v, gstage_v, kstage_v, gob_v, kob_v, sem):
    wid = lax.axis_index("s") * NC + lax.axis_index("c")
    base = wid * BW

    pltpu.sync_copy(uidx.at[pl.ds(wid * 4, 4)], uidx_v)
    cps = [pltpu.async_copy(Eu.at[uidx_v.at[i]],
                            urows_v.at[pl.ds(i * 128, 128)], sem)
           for i in range(4)]
    for c in cps:
        c.wait()
    pltpu.sync_copy(urows_v, feat.at[pl.ds(base, BW), pl.ds(0, 32)])

    pltpu.sync_copy(cidx.at[pl.ds(wid * 4, 4)], cidx_v)
    cps = [pltpu.async_copy(Ecomp.at[cidx_v.at[i]],
                            crows_v.at[pl.ds(i * 128, 128)], sem)
           for i in range(4)]
    for c in cps:
        c.wait()
    pltpu.sync_copy(crows_v, feat.at[pl.ds(base, BW), pl.ds(32, 16)])

    def g_sub(s, carry):
        pltpu.sync_copy(gidx.at[pl.ds(wid * (G_ITER * 8) + s * 8, 8)], gidx_v)
        cps = [pltpu.async_copy(Eg.at[gidx_v.at[i]],
                                gstage_v.at[pl.ds(i * G_CH, G_CH)], sem)
               for i in range(8)]
        for c in cps:
            c.wait()

        def bag(b, c2):
            r0 = b * G_BAG
            a = gstage_v[r0, :]
            for j in range(1, G_BAG):
                a = a + gstage_v[r0 + j, :]
            gob_v[b, :] = a
            return c2
        lax.fori_loop(0, G_SUB, bag, 0)
        pltpu.sync_copy(gob_v,
                        feat.at[pl.ds(base + s * G_SUB, G_SUB), pl.ds(48, 16)])
        return carry
    lax.fori_loop(0, G_ITER, g_sub, 0)

    def k_sub(s, carry):
        pltpu.sync_copy(kidx.at[pl.ds(wid * (K_ITER * 8) + s * 8, 8)], kidx_v)
        cps = [pltpu.async_copy(Ecast.at[kidx_v.at[i]],
                                kstage_v.at[pl.ds(i * K_CH, K_CH)], sem)
               for i in range(8)]
        for c in cps:
            c.wait()

        def bag(b, c2):
            r0 = b * K_BAG
            a0 = kstage_v[r0, pl.ds(0, 16)]
            a1 = kstage_v[r0, pl.ds(16, 16)]
            for j in range(1, K_BAG):
                a0 = a0 + kstage_v[r0 + j, pl.ds(0, 16)]
                a1 = a1 + kstage_v[r0 + j, pl.ds(16, 16)]
            kob_v[b, pl.ds(0, 16)] = a0
            kob_v[b, pl.ds(16, 16)] = a1
            return c2
        lax.fori_loop(0, K_SUB, bag, 0)
        pltpu.sync_copy(kob_v,
                        feat.at[pl.ds(base + s * K_SUB, K_SUB), pl.ds(64, 32)])
        return carry
    lax.fori_loop(0, K_ITER, k_sub, 0)


@functools.cache
def _sc_gather():
    mesh = plsc.VectorSubcoreMesh(core_axis_name="c", subcore_axis_name="s",
                                  num_cores=NC, num_subcores=NS)
    return pl.kernel(
        _sc_body,
        compiler_params=pltpu.CompilerParams(use_tc_tiling_on_sc=False),
        out_type=[
            jax.ShapeDtypeStruct((B, 128), jnp.float32),
        ],
        mesh=mesh,
        scratch_types=[
            pltpu.VMEM((4, 128), jnp.int32),
            pltpu.VMEM((4, 128), jnp.int32),
            pltpu.VMEM((8, G_CH), jnp.int32),
            pltpu.VMEM((8, K_CH), jnp.int32),
            pltpu.VMEM((BW, 32), jnp.float32),
            pltpu.VMEM((BW, 16), jnp.float32),
            pltpu.VMEM((G_SUB * G_BAG, 16), jnp.float32),
            pltpu.VMEM((K_SUB * K_BAG, 32), jnp.float32),
            pltpu.VMEM((G_SUB, 16), jnp.float32),
            pltpu.VMEM((K_SUB, 32), jnp.float32),
            pltpu.SemaphoreType.DMA,
        ],
    )


def _bn(x, g, b):
    m = jnp.mean(x, axis=0)
    xc = x - m
    v = jnp.mean(xc * xc, axis=0)
    return xc * lax.rsqrt(v + 1e-5) * g + b


def _dot(x, wt):
    return jax.lax.dot_general(x, wt, (((1,), (0,)), ((), ())),
                               preferred_element_type=jnp.float32)


def _tc_body(cont, feat,
             Wc0t, bc0, gc0, btc0, Wc1t, bc1, gc1, btc1,
             Wm0t_cat, Wm0t_h, bm0, gm0, btm0,
             Wm1t, bm1, gm1, btm1,
             Wb0t_u, Wb0t_m, bb0, gb0, btb0,
             Wb1t, bb1, gb1, btb1,
             Wat, ba, out):
    x = feat[...]
    h = _bn(jax.nn.relu(_dot(cont[...], Wc0t[...]) + bc0[...]),
            gc0[...], btc0[...])
    h = _bn(jax.nn.relu(_dot(h, Wc1t[...]) + bc1[...]), gc1[...], btc1[...])
    z = _dot(x[:, 32:96], Wm0t_cat[...]) + _dot(h, Wm0t_h[...]) + bm0[...]
    m = _bn(jax.nn.relu(z), gm0[...], btm0[...])
    m = _bn(jax.nn.relu(_dot(m, Wm1t[...]) + bm1[...]), gm1[...], btm1[...])
    z = _dot(x[:, 0:32], Wb0t_u[...]) + _dot(m, Wb0t_m[...]) + bb0[...]
    v = _bn(jax.nn.relu(z), gb0[...], btb0[...])
    v = _bn(jax.nn.relu(_dot(v, Wb1t[...]) + bb1[...]), gb1[...], btb1[...])
    out[...] = jax.nn.sigmoid(_dot(v, Wat[...]) + ba[...])


def kernel(user_indices, cont, genres, cast, comp, Eu, Eg, Ecast, Ecomp,
           Wc0, bc0, gc0, btc0, Wc1, bc1, gc1, btc1,
           Wm0, bm0, gm0, btm0, Wm1, bm1, gm1, btm1,
           Wb0, bb0, gb0, btb0, Wb1, bb1, gb1, btb1,
           Wa, ba):
    uidx = user_indices.astype(jnp.int32).reshape(B // 128, 128)
    cidx = comp.astype(jnp.int32).reshape(B // 128, 128)
    gidx = genres.astype(jnp.int32).reshape(B * G_BAG // G_CH, G_CH)
    kidx = cast.astype(jnp.int32).reshape(B * K_BAG // K_CH, K_CH)

    feat, = _sc_gather()(uidx, cidx, gidx, kidx, Eu, Ecomp, Eg, Ecast)

    out2 = pl.pallas_call(
        _tc_body,
        out_shape=jax.ShapeDtypeStruct((B, 1), jnp.float32),
    )(cont, feat,
      Wc0.T, bc0, gc0, btc0, Wc1.T, bc1, gc1, btc1,
      Wm0[:, 0:64].T, Wm0[:, 64:96].T, bm0, gm0, btm0,
      Wm1.T, bm1, gm1, btm1,
      Wb0[:, 0:32].T, Wb0[:, 32:96].T, bb0, gb0, btb0,
      Wb1.T, bb1, gb1, btb1,
      Wa.T, ba)
    return out2.reshape(-1)
